# Initial kernel scaffold; baseline (speedup 1.0000x reference)
import functools

import jax
import jax.numpy as jnp
from jax import lax
from jax.experimental import pallas as pl
from jax.experimental.pallas import tpu as pltpu
from jax.experimental.pallas import tpu_sc as plsc

BOX = 30.0
RC2 = 9.0

NC = 2
NS = 16
NW = NC * NS
LANES = 16
CHUNK = 2048
IDXROWS = CHUNK // 128


def _lj_body(n_nodes, np_rows, n_chunks,
             pos_hbm, zf_hbm, i_hbm, j_hbm, eps_hbm, sig_hbm,
             fpart_hbm, epart_hbm,
             sh_pos, sh_f, ii, jj, eps_v, sig_v, gi, gj, fi, fj, ev, sem):
    c = lax.axis_index("c")
    s = lax.axis_index("s")
    wid = c * NS + s

    rows = np_rows // NS
    r0 = s * rows
    pltpu.sync_copy(pos_hbm.at[pl.ds(r0, rows)], sh_pos.at[pl.ds(r0, rows)])
    pltpu.sync_copy(zf_hbm.at[pl.ds(r0, rows)], sh_f.at[pl.ds(r0, rows)])

    def zcol(t, _):
        rid = t * LANES + lax.iota(jnp.int32, LANES)
        z = jnp.zeros((LANES,), jnp.float32)
        c3 = jnp.full((LANES,), 3, jnp.int32)
        plsc.store_scatter(fi, [rid, c3], z)
        plsc.store_scatter(fj, [rid, c3], z)
        return 0
    lax.fori_loop(0, CHUNK // LANES, zcol, 0)

    plsc.subcore_barrier()

    row_base = wid * (n_chunks * IDXROWS)

    c0 = jnp.zeros((LANES,), jnp.int32)
    c1 = c0 + 1
    c2 = c0 + 2

    def chunk_body(k, eacc):
        rbase = row_base + k * IDXROWS
        ebase = rbase * 128
        pltpu.sync_copy(i_hbm.at[pl.ds(rbase, IDXROWS)], ii)
        pltpu.sync_copy(j_hbm.at[pl.ds(rbase, IDXROWS)], jj)
        pltpu.sync_copy(eps_hbm.at[pl.ds(ebase, CHUNK)], eps_v)
        pltpu.sync_copy(sig_hbm.at[pl.ds(ebase, CHUNK)], sig_v)
        cps = []
        for g in range(IDXROWS):
            cps.append(pltpu.async_copy(
                sh_pos.at[ii.at[g]], gi.at[pl.ds(g * 128, 128)], sem))
            cps.append(pltpu.async_copy(
                sh_pos.at[jj.at[g]], gj.at[pl.ds(g * 128, 128)], sem))
        for cp in cps:
            cp.wait()

        def grp(t, eacc):
            rid = t * LANES + lax.iota(jnp.int32, LANES)
            xi = plsc.load_gather(gi, [rid, c0])
            yi = plsc.load_gather(gi, [rid, c1])
            zi = plsc.load_gather(gi, [rid, c2])
            xj = plsc.load_gather(gj, [rid, c0])
            yj = plsc.load_gather(gj, [rid, c1])
            zj = plsc.load_gather(gj, [rid, c2])
            dx = xi - xj
            dy = yi - yj
            dz = zi - zj
            dx = dx - jnp.where(dx > 15.0, 30.0, jnp.where(dx < -15.0, -30.0, 0.0))
            dy = dy - jnp.where(dy > 15.0, 30.0, jnp.where(dy < -15.0, -30.0, 0.0))
            dz = dz - jnp.where(dz > 15.0, 30.0, jnp.where(dz < -15.0, -30.0, 0.0))
            r2 = jnp.maximum(dx * dx + dy * dy + dz * dz, 1e-24)
            inv_r2 = 1.0 / r2
            e0 = t * LANES
            ep = eps_v[pl.ds(e0, LANES)]
            sg = sig_v[pl.ds(e0, LANES)]
            s2 = sg * sg * inv_r2
            s6 = s2 * s2 * s2
            s12 = s6 * s6
            mask = r2 < RC2
            u = jnp.where(mask, 4.0 * ep * (s12 - s6), 0.0)
            fsc = jnp.where(mask, 24.0 * ep * inv_r2 * (2.0 * s12 - s6), 0.0)
            fx = fsc * dx
            fy = fsc * dy
            fz = fsc * dz
            plsc.store_scatter(fi, [rid, c0], fx)
            plsc.store_scatter(fi, [rid, c1], fy)
            plsc.store_scatter(fi, [rid, c2], fz)
            plsc.store_scatter(fj, [rid, c0], -fx)
            plsc.store_scatter(fj, [rid, c1], -fy)
            plsc.store_scatter(fj, [rid, c2], -fz)
            return eacc + u

        eacc = lax.fori_loop(0, CHUNK // LANES, grp, eacc)

        cps = []
        for g in range(IDXROWS):
            cps.append(pltpu.async_copy(
                fi.at[pl.ds(g * 128, 128)], sh_f.at[ii.at[g]], sem, add=True))
            cps.append(pltpu.async_copy(
                fj.at[pl.ds(g * 128, 128)], sh_f.at[jj.at[g]], sem, add=True))
        for cp in cps:
            cp.wait()
        return eacc

    eacc = lax.fori_loop(0, n_chunks, chunk_body, jnp.zeros((LANES,), jnp.float32))

    plsc.subcore_barrier()
    pltpu.sync_copy(sh_f.at[pl.ds(r0, rows)],
                    fpart_hbm.at[pl.ds(c * np_rows + r0, rows)])
    ev[...] = eacc
    pltpu.sync_copy(ev, epart_hbm.at[wid])


@jax.jit
def kernel(pos, edge_index, epsilon, sigma):
    n = pos.shape[0]
    e = epsilon.shape[0]
    rows_per_sub = -(-n // NS)
    if (rows_per_sub * 4) % 8:
        rows_per_sub += 1
    np_rows = rows_per_sub * NS
    n_chunks = -(-e // (NW * CHUNK))
    epad = NW * CHUNK * n_chunks

    pos4 = jnp.zeros((np_rows, 4), jnp.float32).at[:n, :3].set(pos)
    zf = jnp.zeros((np_rows, 4), jnp.float32)
    pad = epad - e
    i_p = jnp.concatenate([edge_index[0].astype(jnp.int32),
                           jnp.zeros((pad,), jnp.int32)]).reshape(-1, 128)
    j_p = jnp.concatenate([edge_index[1].astype(jnp.int32),
                           jnp.zeros((pad,), jnp.int32)]).reshape(-1, 128)
    eps_p = jnp.concatenate([epsilon, jnp.zeros((pad,), jnp.float32)])
    sig_p = jnp.concatenate([sigma, jnp.zeros((pad,), jnp.float32)])

    mesh = plsc.VectorSubcoreMesh(core_axis_name="c", subcore_axis_name="s")
    run = pl.kernel(
        functools.partial(_lj_body, n, np_rows, n_chunks),
        out_type=(
            jax.ShapeDtypeStruct((NC * np_rows, 4), jnp.float32),
            jax.ShapeDtypeStruct((NW, LANES), jnp.float32),
        ),
        mesh=mesh,
        scratch_types=[
            pltpu.VMEM_SHARED((np_rows, 4), jnp.float32),
            pltpu.VMEM_SHARED((np_rows, 4), jnp.float32),
            pltpu.VMEM((IDXROWS, 128), jnp.int32),
            pltpu.VMEM((IDXROWS, 128), jnp.int32),
            pltpu.VMEM((CHUNK,), jnp.float32),
            pltpu.VMEM((CHUNK,), jnp.float32),
            pltpu.VMEM((CHUNK, 4), jnp.float32),
            pltpu.VMEM((CHUNK, 4), jnp.float32),
            pltpu.VMEM((CHUNK, 4), jnp.float32),
            pltpu.VMEM((CHUNK, 4), jnp.float32),
            pltpu.VMEM((LANES,), jnp.float32),
            pltpu.SemaphoreType.DMA,
        ],
    )
    fpart, epart = run(pos4, zf, i_p, j_p, eps_p, sig_p)
    forces = fpart[:n, :3] + fpart[np_rows:np_rows + n, :3]
    total_energy = jnp.sum(epart)
    return (total_energy, forces)


# speedup vs baseline: 21.3899x; 21.3899x over previous
import functools

import jax
import jax.numpy as jnp
from jax import lax
from jax.experimental import pallas as pl
from jax.experimental.pallas import tpu as pltpu
from jax.experimental.pallas import tpu_sc as plsc

NC = 2
NS = 16
NW = NC * NS
LANES = 16
CHUNK = 2048
IDXROWS = CHUNK // 128


def _lj_body(n_nodes, np_rows, n_chunks,
             px_hbm, py_hbm, pz_hbm, i_hbm, j_hbm, eps_hbm, sig_hbm,
             fpart_hbm, epart_hbm,
             sh_x, sh_y, sh_z, sh_fx, sh_fy, sh_fz,
             ii, jj, eps_v, sig_v,
             xi, yi, zi, xj, yj, zj,
             fxi, fyi, fzi, fxj, fyj, fzj, ev, sem):
    c = lax.axis_index("c")
    s = lax.axis_index("s")
    wid = c * NS + s

    rows = np_rows // NS
    r0 = s * rows
    pieces = []
    off = 0
    while off < rows:
        pieces.append((off, min(CHUNK, rows - off)))
        off += CHUNK
    for hbm_ref, sh_ref in ((px_hbm, sh_x), (py_hbm, sh_y), (pz_hbm, sh_z)):
        for (o, ln) in pieces:
            pltpu.sync_copy(hbm_ref.at[pl.ds(r0 + o, ln)], xi.at[pl.ds(0, ln)])
            pltpu.sync_copy(xi.at[pl.ds(0, ln)], sh_ref.at[pl.ds(r0 + o, ln)])

    def zbuf(t, _):
        fxi[pl.ds(t * LANES, LANES)] = jnp.zeros((LANES,), jnp.float32)
        return 0
    lax.fori_loop(0, CHUNK // LANES, zbuf, 0)
    for sh_ref in (sh_fx, sh_fy, sh_fz):
        for (o, ln) in pieces:
            pltpu.sync_copy(fxi.at[pl.ds(0, ln)], sh_ref.at[pl.ds(r0 + o, ln)])
    plsc.subcore_barrier()

    row_base = wid * (n_chunks * IDXROWS)

    def chunk_body(k, eacc):
        rbase = row_base + k * IDXROWS
        ebase = rbase * 128
        pltpu.sync_copy(i_hbm.at[pl.ds(rbase, IDXROWS)], ii)
        pltpu.sync_copy(j_hbm.at[pl.ds(rbase, IDXROWS)], jj)
        pltpu.sync_copy(eps_hbm.at[pl.ds(ebase, CHUNK)], eps_v)
        pltpu.sync_copy(sig_hbm.at[pl.ds(ebase, CHUNK)], sig_v)
        cps = []
        for g in range(IDXROWS):
            dsl = pl.ds(g * 128, 128)
            ig = ii.at[g]
            jg = jj.at[g]
            cps.append(pltpu.async_copy(sh_x.at[ig], xi.at[dsl], sem))
            cps.append(pltpu.async_copy(sh_y.at[ig], yi.at[dsl], sem))
            cps.append(pltpu.async_copy(sh_z.at[ig], zi.at[dsl], sem))
            cps.append(pltpu.async_copy(sh_x.at[jg], xj.at[dsl], sem))
            cps.append(pltpu.async_copy(sh_y.at[jg], yj.at[dsl], sem))
            cps.append(pltpu.async_copy(sh_z.at[jg], zj.at[dsl], sem))
        for cp in cps:
            cp.wait()

        def grp(t, eacc):
            vs = pl.ds(t * LANES, LANES)
            dx = xi[vs] - xj[vs]
            dy = yi[vs] - yj[vs]
            dz = zi[vs] - zj[vs]
            dx = dx - jnp.where(dx > 15.0, 30.0, jnp.where(dx < -15.0, -30.0, 0.0))
            dy = dy - jnp.where(dy > 15.0, 30.0, jnp.where(dy < -15.0, -30.0, 0.0))
            dz = dz - jnp.where(dz > 15.0, 30.0, jnp.where(dz < -15.0, -30.0, 0.0))
            r2 = jnp.maximum(dx * dx + dy * dy + dz * dz, 1e-24)
            inv_r2 = 1.0 / r2
            ep = eps_v[vs]
            sg = sig_v[vs]
            s2 = sg * sg * inv_r2
            s6 = s2 * s2 * s2
            s12 = s6 * s6
            mask = r2 < 9.0
            u = jnp.where(mask, 4.0 * ep * (s12 - s6), 0.0)
            fsc = jnp.where(mask, 24.0 * ep * inv_r2 * (2.0 * s12 - s6), 0.0)
            fx = fsc * dx
            fy = fsc * dy
            fz = fsc * dz
            fxi[vs] = fx
            fyi[vs] = fy
            fzi[vs] = fz
            fxj[vs] = -fx
            fyj[vs] = -fy
            fzj[vs] = -fz
            return eacc + u

        eacc = lax.fori_loop(0, CHUNK // LANES, grp, eacc)

        cps = []
        for g in range(IDXROWS):
            dsl = pl.ds(g * 128, 128)
            ig = ii.at[g]
            jg = jj.at[g]
            cps.append(pltpu.async_copy(fxi.at[dsl], sh_fx.at[ig], sem, add=True))
            cps.append(pltpu.async_copy(fyi.at[dsl], sh_fy.at[ig], sem, add=True))
            cps.append(pltpu.async_copy(fzi.at[dsl], sh_fz.at[ig], sem, add=True))
            cps.append(pltpu.async_copy(fxj.at[dsl], sh_fx.at[jg], sem, add=True))
            cps.append(pltpu.async_copy(fyj.at[dsl], sh_fy.at[jg], sem, add=True))
            cps.append(pltpu.async_copy(fzj.at[dsl], sh_fz.at[jg], sem, add=True))
        for cp in cps:
            cp.wait()
        return eacc

    eacc = lax.fori_loop(0, n_chunks, chunk_body, jnp.zeros((LANES,), jnp.float32))

    plsc.subcore_barrier()
    base = c * 3 * np_rows
    for comp, sh_ref in enumerate((sh_fx, sh_fy, sh_fz)):
        for (o, ln) in pieces:
            pltpu.sync_copy(sh_ref.at[pl.ds(r0 + o, ln)], xi.at[pl.ds(0, ln)])
            pltpu.sync_copy(xi.at[pl.ds(0, ln)],
                            fpart_hbm.at[pl.ds(base + comp * np_rows + r0 + o, ln)])
    ev[...] = eacc
    pltpu.sync_copy(ev, epart_hbm.at[pl.ds(wid * LANES, LANES)])


@jax.jit
def kernel(pos, edge_index, epsilon, sigma):
    n = pos.shape[0]
    e = epsilon.shape[0]
    rows_per_sub = -(-n // (NS * 8)) * 8
    np_rows = rows_per_sub * NS
    n_chunks = -(-e // (NW * CHUNK))
    epad = NW * CHUNK * n_chunks

    pz3 = jnp.zeros((np_rows - n,), jnp.float32)
    px = jnp.concatenate([pos[:, 0], pz3])
    py = jnp.concatenate([pos[:, 1], pz3])
    pz = jnp.concatenate([pos[:, 2], pz3])
    pad = epad - e
    i_p = jnp.concatenate([edge_index[0].astype(jnp.int32),
                           jnp.zeros((pad,), jnp.int32)]).reshape(-1, 128)
    j_p = jnp.concatenate([edge_index[1].astype(jnp.int32),
                           jnp.zeros((pad,), jnp.int32)]).reshape(-1, 128)
    eps_p = jnp.concatenate([epsilon, jnp.zeros((pad,), jnp.float32)])
    sig_p = jnp.concatenate([sigma, jnp.zeros((pad,), jnp.float32)])

    mesh = plsc.VectorSubcoreMesh(core_axis_name="c", subcore_axis_name="s")
    run = pl.kernel(
        functools.partial(_lj_body, n, np_rows, n_chunks),
        out_type=(
            jax.ShapeDtypeStruct((NC * 3 * np_rows,), jnp.float32),
            jax.ShapeDtypeStruct((NW * LANES,), jnp.float32),
        ),
        mesh=mesh,
        scratch_types=(
            [pltpu.VMEM_SHARED((np_rows,), jnp.float32) for _ in range(6)]
            + [pltpu.VMEM((IDXROWS, 128), jnp.int32) for _ in range(2)]
            + [pltpu.VMEM((CHUNK,), jnp.float32) for _ in range(14)]
            + [pltpu.VMEM((LANES,), jnp.float32),
               pltpu.SemaphoreType.DMA]
        ),
    )
    fpart, epart = run(px, py, pz, i_p, j_p, eps_p, sig_p)
    fp = fpart.reshape(NC, 3, np_rows)
    forces = (fp[0] + fp[1])[:, :n].T
    total_energy = jnp.sum(epart)
    return (total_energy, forces)
